# baseline (speedup 1.0000x reference)
import functools

import jax
import jax.numpy as jnp
from jax.experimental import pallas as pl
from jax.experimental.pallas import tpu as pltpu

B, S, H = 4, 2048, 1024
E, DFF = 8, 2048
COEF = 0.01

ROUTER_TS = 256
DFF_TK = 512


def _router_kernel(x_ref, gamma_ref, beta_ref, rw_ref, rb_ref,
                   idx_ref, loss_ref, acc_ref):
    t = pl.program_id(0)
    nt = pl.num_programs(0)

    @pl.when(t == 0)
    def _init():
        acc_ref[...] = jnp.zeros_like(acc_ref)

    acc_ref[...] += jnp.sum(x_ref[...], axis=1)

    @pl.when(t == nt - 1)
    def _finish():
        mean = acc_ref[...] * (1.0 / S)
        mu = jnp.mean(mean, axis=-1, keepdims=True)
        var = jnp.mean((mean - mu) ** 2, axis=-1, keepdims=True)
        nrm = (mean - mu) * jax.lax.rsqrt(var + 1e-5)
        nrm = nrm * gamma_ref[...] + beta_ref[...]
        logits = jnp.dot(nrm, rw_ref[...],
                         preferred_element_type=jnp.float32) + rb_ref[...]
        lmax = jnp.max(logits, axis=-1, keepdims=True)
        ex = jnp.exp(logits - lmax)
        probs = ex / jnp.sum(ex, axis=-1, keepdims=True)
        eids = jax.lax.broadcasted_iota(jnp.int32, logits.shape, 1)
        cand = jnp.where(logits >= lmax, eids, E)
        idx_ref[...] = jnp.min(cand, axis=-1, keepdims=True)
        avg = jnp.mean(probs, axis=0, keepdims=True)
        loss = jnp.mean((avg - 1.0 / E) ** 2, keepdims=True)
        loss_ref[...] = COEF * loss


def _ffn_kernel(idx_ref, x_ref, w1_ref, b1_ref, w2_ref, b2_ref, o_ref,
                xb_ref):
    k = pl.program_id(1)

    @pl.when(k == 0)
    def _stage_x():
        xb_ref[...] = x_ref[0].astype(jnp.bfloat16)
        o_ref[0] = jnp.broadcast_to(b2_ref[0], o_ref.shape[1:])

    h = jnp.dot(xb_ref[...], w1_ref[0].astype(jnp.bfloat16),
                preferred_element_type=jnp.float32)
    h = jax.nn.gelu(h + b1_ref[0])
    o_ref[0] += jnp.dot(h.astype(jnp.bfloat16),
                        w2_ref[0].astype(jnp.bfloat16),
                        preferred_element_type=jnp.float32)


@jax.jit
def kernel(inputs, ln_gamma, ln_beta, router_w, router_b, W1, b1, W2, b2,
           current_depth):
    del current_depth
    gamma2 = ln_gamma.reshape(1, H)
    beta2 = ln_beta.reshape(1, H)
    rb2 = router_b.reshape(1, E)

    nt = S // ROUTER_TS
    idx2, loss2 = pl.pallas_call(
        _router_kernel,
        grid=(nt,),
        in_specs=[
            pl.BlockSpec((B, ROUTER_TS, H), lambda t: (0, t, 0)),
            pl.BlockSpec((1, H), lambda t: (0, 0)),
            pl.BlockSpec((1, H), lambda t: (0, 0)),
            pl.BlockSpec((H, E), lambda t: (0, 0)),
            pl.BlockSpec((1, E), lambda t: (0, 0)),
        ],
        out_specs=[
            pl.BlockSpec((B, 1), lambda t: (0, 0)),
            pl.BlockSpec((1, 1), lambda t: (0, 0)),
        ],
        out_shape=[
            jax.ShapeDtypeStruct((B, 1), jnp.int32),
            jax.ShapeDtypeStruct((1, 1), jnp.float32),
        ],
        scratch_shapes=[pltpu.VMEM((B, H), jnp.float32)],
    )(inputs, gamma2, beta2, router_w, rb2)

    expert_idx = idx2.reshape(B)
    total_aux_loss = loss2[0, 0]

    nk = DFF // DFF_TK
    grid_spec = pltpu.PrefetchScalarGridSpec(
        num_scalar_prefetch=1,
        grid=(B, nk),
        in_specs=[
            pl.BlockSpec((1, S, H), lambda b, k, idx: (b, 0, 0)),
            pl.BlockSpec((1, H, DFF_TK), lambda b, k, idx: (idx[b], 0, k)),
            pl.BlockSpec((1, 1, DFF_TK), lambda b, k, idx: (idx[b], 0, k)),
            pl.BlockSpec((1, DFF_TK, H), lambda b, k, idx: (idx[b], k, 0)),
            pl.BlockSpec((1, 1, H), lambda b, k, idx: (idx[b], 0, 0)),
        ],
        out_specs=pl.BlockSpec((1, S, H), lambda b, k, idx: (b, 0, 0)),
        scratch_shapes=[
            pltpu.VMEM((S, H), jnp.bfloat16),
        ],
    )
    output = pl.pallas_call(
        _ffn_kernel,
        grid_spec=grid_spec,
        out_shape=jax.ShapeDtypeStruct((B, S, H), jnp.float32),
    )(expert_idx, inputs, W1, b1.reshape(E, 1, DFF), W2, b2.reshape(E, 1, H))

    return (output, total_aux_loss)


# speedup vs baseline: 1.1753x; 1.1753x over previous
import functools

import jax
import jax.numpy as jnp
from jax.experimental import pallas as pl
from jax.experimental.pallas import tpu as pltpu

B, S, H = 4, 2048, 1024
E, DFF = 8, 2048
COEF = 0.01

ROUTER_TS = 256
FFN_TS = 1024


def _router_kernel(x_ref, gamma_ref, beta_ref, rw_ref, rb_ref,
                   idx_ref, loss_ref, acc_ref):
    t = pl.program_id(0)
    nt = pl.num_programs(0)

    @pl.when(t == 0)
    def _init():
        acc_ref[...] = jnp.zeros_like(acc_ref)

    acc_ref[...] += jnp.sum(x_ref[...], axis=1)

    @pl.when(t == nt - 1)
    def _finish():
        mean = acc_ref[...] * (1.0 / S)
        mu = jnp.mean(mean, axis=-1, keepdims=True)
        var = jnp.mean((mean - mu) ** 2, axis=-1, keepdims=True)
        nrm = (mean - mu) * jax.lax.rsqrt(var + 1e-5)
        nrm = nrm * gamma_ref[...] + beta_ref[...]
        logits = jnp.dot(nrm, rw_ref[...],
                         preferred_element_type=jnp.float32) + rb_ref[...]
        lmax = jnp.max(logits, axis=-1, keepdims=True)
        ex = jnp.exp(logits - lmax)
        probs = ex / jnp.sum(ex, axis=-1, keepdims=True)
        eids = jax.lax.broadcasted_iota(jnp.int32, logits.shape, 1)
        cand = jnp.where(logits >= lmax, eids, E)
        idx_ref[...] = jnp.min(cand, axis=-1, keepdims=True)
        avg = jnp.mean(probs, axis=0, keepdims=True)
        loss = jnp.mean((avg - 1.0 / E) ** 2, keepdims=True)
        loss_ref[...] = COEF * loss


def _ffn_kernel(idx_ref, x_ref, w1_ref, b1_ref, w2_ref, b2_ref, o_ref):
    x = x_ref[0]
    h = jnp.dot(x, w1_ref[0], preferred_element_type=jnp.float32)
    h = jax.nn.gelu(h + b1_ref[0])
    o = jnp.dot(h, w2_ref[0], preferred_element_type=jnp.float32)
    o_ref[0] = o + b2_ref[0]


@jax.jit
def kernel(inputs, ln_gamma, ln_beta, router_w, router_b, W1, b1, W2, b2,
           current_depth):
    del current_depth
    gamma2 = ln_gamma.reshape(1, H)
    beta2 = ln_beta.reshape(1, H)
    rb2 = router_b.reshape(1, E)

    nt = S // ROUTER_TS
    idx2, loss2 = pl.pallas_call(
        _router_kernel,
        grid=(nt,),
        in_specs=[
            pl.BlockSpec((B, ROUTER_TS, H), lambda t: (0, t, 0)),
            pl.BlockSpec((1, H), lambda t: (0, 0)),
            pl.BlockSpec((1, H), lambda t: (0, 0)),
            pl.BlockSpec((H, E), lambda t: (0, 0)),
            pl.BlockSpec((1, E), lambda t: (0, 0)),
        ],
        out_specs=[
            pl.BlockSpec((B, 1), lambda t: (0, 0)),
            pl.BlockSpec((1, 1), lambda t: (0, 0)),
        ],
        out_shape=[
            jax.ShapeDtypeStruct((B, 1), jnp.int32),
            jax.ShapeDtypeStruct((1, 1), jnp.float32),
        ],
        scratch_shapes=[pltpu.VMEM((B, H), jnp.float32)],
    )(inputs, gamma2, beta2, router_w, rb2)

    expert_idx = idx2.reshape(B)
    total_aux_loss = loss2[0, 0]

    ns = S // FFN_TS
    grid_spec = pltpu.PrefetchScalarGridSpec(
        num_scalar_prefetch=1,
        grid=(B, ns),
        in_specs=[
            pl.BlockSpec((1, FFN_TS, H), lambda b, s, idx: (b, s, 0)),
            pl.BlockSpec((1, H, DFF), lambda b, s, idx: (idx[b], 0, 0)),
            pl.BlockSpec((1, 1, DFF), lambda b, s, idx: (idx[b], 0, 0)),
            pl.BlockSpec((1, DFF, H), lambda b, s, idx: (idx[b], 0, 0)),
            pl.BlockSpec((1, 1, H), lambda b, s, idx: (idx[b], 0, 0)),
        ],
        out_specs=pl.BlockSpec((1, FFN_TS, H), lambda b, s, idx: (b, s, 0)),
    )
    output = pl.pallas_call(
        _ffn_kernel,
        grid_spec=grid_spec,
        out_shape=jax.ShapeDtypeStruct((B, S, H), jnp.float32),
    )(expert_idx, inputs, W1, b1.reshape(E, 1, DFF), W2, b2.reshape(E, 1, H))

    return (output, total_aux_loss)
